# baseline (speedup 1.0000x reference)
import jax
import jax.numpy as jnp
from jax.experimental import pallas as pl

N_GROUP = 8
TOPK_GROUP = 3
TOP_K = 8


def _router_combine(logits):
    m, e = logits.shape
    gsz = e // N_GROUP
    scores = jax.nn.softmax(logits, axis=-1)

    gi = jax.lax.broadcasted_iota(jnp.int32, (N_GROUP, e), 0)
    ge = jax.lax.broadcasted_iota(jnp.int32, (N_GROUP, e), 1) // gsz
    mem = ge == gi

    gs = jnp.max(
        jnp.where(mem[None, :, :], scores[:, None, :], -jnp.inf), axis=-1
    )

    cnt_g = jnp.sum(
        (gs[:, None, :] > gs[:, :, None]).astype(jnp.float32), axis=-1
    )
    selg = (cnt_g < TOPK_GROUP).astype(jnp.float32)

    sel_full = jnp.max(
        jnp.where(mem[None, :, :], selg[:, :, None], 0.0), axis=1
    )
    tmp = jnp.where(sel_full > 0.0, scores, 0.0)

    cnt_e = jnp.sum(
        (tmp[:, None, :] > tmp[:, :, None]).astype(jnp.float32), axis=-1
    )
    return jnp.where(cnt_e < TOP_K, tmp, 0.0)


def _shared_router_body(x_ref, sgu_ref, sdn_ref, gw_ref, so_ref, comb_ref):
    x = x_ref[...]
    xb = x.astype(jnp.bfloat16)
    ff2 = sgu_ref.shape[0] // 2

    sh = jax.lax.dot_general(
        xb, sgu_ref[...].astype(jnp.bfloat16),
        (((1,), (1,)), ((), ())), preferred_element_type=jnp.float32,
    )
    act = (jax.nn.silu(sh[:, :ff2]) * sh[:, ff2:]).astype(jnp.bfloat16)
    so_ref[...] = jax.lax.dot_general(
        act, sdn_ref[...].astype(jnp.bfloat16),
        (((1,), (1,)), ((), ())), preferred_element_type=jnp.float32,
    )

    logits = jax.lax.dot_general(
        x, gw_ref[...], (((1,), (1,)), ((), ())),
        preferred_element_type=jnp.float32,
    )
    comb_ref[...] = _router_combine(logits)


EXP_PER_STEP = 4


def _expert_body(xb_ref, comb_ref, shared_ref, gu_ref, dn_ref, out_ref):
    i = pl.program_id(0)
    ff = dn_ref.shape[2]
    comb = comb_ref[...]
    lane = jax.lax.broadcasted_iota(jnp.int32, comb.shape, 1)

    contrib = None
    for j in range(EXP_PER_STEP):
        e = i * EXP_PER_STEP + j
        gu = gu_ref[j].astype(jnp.bfloat16)
        h = jax.lax.dot_general(
            xb_ref[...], gu, (((1,), (1,)), ((), ())),
            preferred_element_type=jnp.float32,
        )
        act = jax.nn.silu(h[:, :ff]) * h[:, ff:]

        col = jnp.sum(jnp.where(lane == e, comb, 0.0), axis=1, keepdims=True)
        actw = (act * col).astype(jnp.bfloat16)
        c = jax.lax.dot_general(
            actw, dn_ref[j].astype(jnp.bfloat16),
            (((1,), (1,)), ((), ())), preferred_element_type=jnp.float32,
        )
        contrib = c if contrib is None else contrib + c

    @pl.when(i == 0)
    def _():
        out_ref[...] = shared_ref[...] + contrib

    @pl.when(i > 0)
    def _():
        out_ref[...] = out_ref[...] + contrib


def kernel(x, gate_w, gate_up_proj, down_proj, shared_gate_up, shared_down):
    m, hidden = x.shape
    n_e, two_ff, _ = gate_up_proj.shape
    ff = down_proj.shape[2]

    shared_out, combine = pl.pallas_call(
        _shared_router_body,
        out_shape=(
            jax.ShapeDtypeStruct((m, hidden), jnp.float32),
            jax.ShapeDtypeStruct((m, n_e), jnp.float32),
        ),
    )(x, shared_gate_up, shared_down, gate_w)

    xb = x.astype(jnp.bfloat16)
    out = pl.pallas_call(
        _expert_body,
        grid=(n_e // EXP_PER_STEP,),
        in_specs=[
            pl.BlockSpec((m, hidden), lambda e: (0, 0)),
            pl.BlockSpec((m, n_e), lambda e: (0, 0)),
            pl.BlockSpec((m, hidden), lambda e: (0, 0)),
            pl.BlockSpec((EXP_PER_STEP, two_ff, hidden), lambda e: (e, 0, 0)),
            pl.BlockSpec((EXP_PER_STEP, hidden, ff), lambda e: (e, 0, 0)),
        ],
        out_specs=pl.BlockSpec((m, hidden), lambda e: (0, 0)),
        out_shape=jax.ShapeDtypeStruct((m, hidden), jnp.float32),
    )(xb, combine, shared_out, gate_up_proj, down_proj)
    return out


# speedup vs baseline: 1.0427x; 1.0427x over previous
import jax
import jax.numpy as jnp
from jax.experimental import pallas as pl

N_GROUP = 8
TOPK_GROUP = 3
TOP_K = 8


def _router_combine(logits):
    m, e = logits.shape
    gsz = e // N_GROUP
    scores = jax.nn.softmax(logits, axis=-1)

    gi = jax.lax.broadcasted_iota(jnp.int32, (N_GROUP, e), 0)
    ge = jax.lax.broadcasted_iota(jnp.int32, (N_GROUP, e), 1) // gsz
    mem = ge == gi

    gs = jnp.max(
        jnp.where(mem[None, :, :], scores[:, None, :], -jnp.inf), axis=-1
    )

    cnt_g = jnp.sum(
        (gs[:, None, :] > gs[:, :, None]).astype(jnp.float32), axis=-1
    )
    selg = (cnt_g < TOPK_GROUP).astype(jnp.float32)

    sel_full = jnp.max(
        jnp.where(mem[None, :, :], selg[:, :, None], 0.0), axis=1
    )
    tmp = jnp.where(sel_full > 0.0, scores, 0.0)

    cnt_e = jnp.sum(
        (tmp[:, None, :] > tmp[:, :, None]).astype(jnp.float32), axis=-1
    )
    return jnp.where(cnt_e < TOP_K, tmp, 0.0)


def _shared_router_body(x_ref, sgu_ref, sdn_ref, gw_ref, so_ref, comb_ref):
    x = x_ref[...]
    xb = x.astype(jnp.bfloat16)
    ff2 = sgu_ref.shape[0] // 2

    sh = jax.lax.dot_general(
        xb, sgu_ref[...].astype(jnp.bfloat16),
        (((1,), (1,)), ((), ())), preferred_element_type=jnp.float32,
    )
    act = (jax.nn.silu(sh[:, :ff2]) * sh[:, ff2:]).astype(jnp.bfloat16)
    so_ref[...] = jax.lax.dot_general(
        act, sdn_ref[...].astype(jnp.bfloat16),
        (((1,), (1,)), ((), ())), preferred_element_type=jnp.float32,
    )

    logits = jax.lax.dot_general(
        x, gw_ref[...], (((1,), (1,)), ((), ())),
        preferred_element_type=jnp.float32,
    )
    comb_ref[...] = _router_combine(logits)


EXP_PER_STEP = 2


def _expert_body(xb_ref, comb_ref, shared_ref, g_ref, u_ref, dn_ref, out_ref):
    i = pl.program_id(0)
    comb = comb_ref[...]
    lane = jax.lax.broadcasted_iota(jnp.int32, comb.shape, 1)

    contrib = None
    for j in range(EXP_PER_STEP):
        e = i * EXP_PER_STEP + j
        xb = xb_ref[...]
        hg = jax.lax.dot_general(
            xb, g_ref[j].astype(jnp.bfloat16), (((1,), (1,)), ((), ())),
            preferred_element_type=jnp.float32,
        )
        hu = jax.lax.dot_general(
            xb, u_ref[j].astype(jnp.bfloat16), (((1,), (1,)), ((), ())),
            preferred_element_type=jnp.float32,
        )
        act = jax.nn.silu(hg) * hu

        col = jnp.sum(jnp.where(lane == e, comb, 0.0), axis=1, keepdims=True)
        actw = (act * col).astype(jnp.bfloat16)
        c = jax.lax.dot_general(
            actw, dn_ref[j].astype(jnp.bfloat16),
            (((1,), (1,)), ((), ())), preferred_element_type=jnp.float32,
        )
        contrib = c if contrib is None else contrib + c

    @pl.when(i == 0)
    def _():
        out_ref[...] = shared_ref[...] + contrib

    @pl.when(i > 0)
    def _():
        out_ref[...] = out_ref[...] + contrib


def kernel(x, gate_w, gate_up_proj, down_proj, shared_gate_up, shared_down):
    m, hidden = x.shape
    n_e, two_ff, _ = gate_up_proj.shape
    ff = down_proj.shape[2]

    shared_out, combine = pl.pallas_call(
        _shared_router_body,
        out_shape=(
            jax.ShapeDtypeStruct((m, hidden), jnp.float32),
            jax.ShapeDtypeStruct((m, n_e), jnp.float32),
        ),
    )(x, shared_gate_up, shared_down, gate_w)

    xb = x.astype(jnp.bfloat16)
    out = pl.pallas_call(
        _expert_body,
        grid=(n_e // EXP_PER_STEP,),
        in_specs=[
            pl.BlockSpec((m, hidden), lambda e: (0, 0)),
            pl.BlockSpec((m, n_e), lambda e: (0, 0)),
            pl.BlockSpec((m, hidden), lambda e: (0, 0)),
            pl.BlockSpec((EXP_PER_STEP, two_ff // 2, hidden), lambda e: (e, 0, 0)),
            pl.BlockSpec((EXP_PER_STEP, two_ff // 2, hidden), lambda e: (e, 1, 0)),
            pl.BlockSpec((EXP_PER_STEP, hidden, ff), lambda e: (e, 0, 0)),
        ],
        out_specs=pl.BlockSpec((m, hidden), lambda e: (0, 0)),
        out_shape=jax.ShapeDtypeStruct((m, hidden), jnp.float32),
    )(xb, combine, shared_out, gate_up_proj, gate_up_proj, down_proj)
    return out


# speedup vs baseline: 1.0672x; 1.0235x over previous
import jax
import jax.numpy as jnp
from jax.experimental import pallas as pl
from jax.experimental.pallas import tpu as pltpu

N_GROUP = 8
TOPK_GROUP = 3
TOP_K = 8
EXP_PER_STEP = 2


def _router_combine(logits):
    m, e = logits.shape
    gsz = e // N_GROUP
    scores = jax.nn.softmax(logits, axis=-1)

    gi = jax.lax.broadcasted_iota(jnp.int32, (N_GROUP, e), 0)
    ge = jax.lax.broadcasted_iota(jnp.int32, (N_GROUP, e), 1) // gsz
    mem = ge == gi

    gs = jnp.max(
        jnp.where(mem[None, :, :], scores[:, None, :], -jnp.inf), axis=-1
    )

    cnt_g = jnp.sum(
        (gs[:, None, :] > gs[:, :, None]).astype(jnp.float32), axis=-1
    )
    selg = (cnt_g < TOPK_GROUP).astype(jnp.float32)

    sel_full = jnp.max(
        jnp.where(mem[None, :, :], selg[:, :, None], 0.0), axis=1
    )
    tmp = jnp.where(sel_full > 0.0, scores, 0.0)

    cnt_e = jnp.sum(
        (tmp[:, None, :] > tmp[:, :, None]).astype(jnp.float32), axis=-1
    )
    return jnp.where(cnt_e < TOP_K, tmp, 0.0)


def _shared_body(x_ref, sgu_ref, sdn_ref, so_ref):
    xb = x_ref[...].astype(jnp.bfloat16)
    ff2 = sgu_ref.shape[0] // 2
    sh = jax.lax.dot_general(
        xb, sgu_ref[...].astype(jnp.bfloat16),
        (((1,), (1,)), ((), ())), preferred_element_type=jnp.float32,
    )
    act = (jax.nn.silu(sh[:, :ff2]) * sh[:, ff2:]).astype(jnp.bfloat16)
    so_ref[...] = jax.lax.dot_general(
        act, sdn_ref[...].astype(jnp.bfloat16),
        (((1,), (1,)), ((), ())), preferred_element_type=jnp.float32,
    )


def _expert_body(x_ref, gw_ref, shared_ref, g_ref, u_ref, dn_ref, out_ref,
                 comb_ref):
    i = pl.program_id(0)
    x = x_ref[...]
    xb = x.astype(jnp.bfloat16)

    @pl.when(i == 0)
    def _():
        logits = jax.lax.dot_general(
            x, gw_ref[...], (((1,), (1,)), ((), ())),
            preferred_element_type=jnp.float32,
        )
        comb_ref[...] = _router_combine(logits)

    comb = comb_ref[...]
    lane = jax.lax.broadcasted_iota(jnp.int32, comb.shape, 1)

    contrib = None
    for j in range(EXP_PER_STEP):
        e = i * EXP_PER_STEP + j
        hg = jax.lax.dot_general(
            xb, g_ref[j].astype(jnp.bfloat16), (((1,), (1,)), ((), ())),
            preferred_element_type=jnp.float32,
        )
        hu = jax.lax.dot_general(
            xb, u_ref[j].astype(jnp.bfloat16), (((1,), (1,)), ((), ())),
            preferred_element_type=jnp.float32,
        )
        act = jax.nn.silu(hg) * hu

        col = jnp.sum(jnp.where(lane == e, comb, 0.0), axis=1, keepdims=True)
        actw = (act * col).astype(jnp.bfloat16)
        c = jax.lax.dot_general(
            actw, dn_ref[j].astype(jnp.bfloat16),
            (((1,), (1,)), ((), ())), preferred_element_type=jnp.float32,
        )
        contrib = c if contrib is None else contrib + c

    @pl.when(i == 0)
    def _():
        out_ref[...] = shared_ref[...] + contrib

    @pl.when(i > 0)
    def _():
        out_ref[...] = out_ref[...] + contrib


def kernel(x, gate_w, gate_up_proj, down_proj, shared_gate_up, shared_down):
    m, hidden = x.shape
    n_e, two_ff, _ = gate_up_proj.shape
    ff = down_proj.shape[2]

    shared_out = pl.pallas_call(
        _shared_body,
        out_shape=jax.ShapeDtypeStruct((m, hidden), jnp.float32),
    )(x, shared_gate_up, shared_down)

    out = pl.pallas_call(
        _expert_body,
        grid=(n_e // EXP_PER_STEP,),
        in_specs=[
            pl.BlockSpec((m, hidden), lambda e: (0, 0)),
            pl.BlockSpec((n_e, hidden), lambda e: (0, 0)),
            pl.BlockSpec((m, hidden), lambda e: (0, 0)),
            pl.BlockSpec((EXP_PER_STEP, two_ff // 2, hidden), lambda e: (e, 0, 0)),
            pl.BlockSpec((EXP_PER_STEP, two_ff // 2, hidden), lambda e: (e, 1, 0)),
            pl.BlockSpec((EXP_PER_STEP, hidden, ff), lambda e: (e, 0, 0)),
        ],
        out_specs=pl.BlockSpec((m, hidden), lambda e: (0, 0)),
        out_shape=jax.ShapeDtypeStruct((m, hidden), jnp.float32),
        scratch_shapes=[pltpu.VMEM((m, n_e), jnp.float32)],
    )(x, gate_w, shared_out, gate_up_proj, gate_up_proj, down_proj)
    return out
